# baseline (speedup 1.0000x reference)
import functools

import jax
import jax.numpy as jnp
from jax.experimental import pallas as pl
from jax.experimental.pallas import tpu as pltpu

_K = 64


def _fold_count(m16):
    a = m16
    n = a.shape[0]
    while n > 1:
        d = next((p for p in (2, 3, 5, 7, 11, 13) if n % p == 0), n)
        step = n // d
        acc = a[0:step]
        for j in range(1, d):
            acc = acc + a[j * step:(j + 1) * step]
        a = acc
        n = step
    return jnp.sum(a[0].astype(jnp.int32))


def _bcast16(t32, lanes):
    return jnp.full((1, 16, lanes), t32, jnp.int32).astype(jnp.int16)


def _count_ge(v16, t32):
    m16 = jnp.where(v16 >= _bcast16(t32, v16.shape[-1]),
                    jnp.int16(1), jnp.int16(0))
    return _fold_count(m16)


def _search16(v16, cand0_if_neg, k, nbits=15):
    nonneg = _count_ge(v16, jnp.int32(0))
    cand = jnp.where(nonneg >= k, jnp.int32(0), cand0_if_neg)
    for b in range(nbits - 1, -1, -1):
        t = cand | jnp.int32(1 << b)
        cnt = _count_ge(v16, t)
        cand = jnp.where(cnt >= k, t, cand)
    return cand


def _topk_mask_kernel(x_ref, o_ref, *, k, fold):
    xb4 = x_ref[0]
    w = xb4.shape[-1]
    rows = xb4.size // w
    xb = xb4.reshape(rows, w)
    half = rows // fold
    xp = jnp.concatenate([xb[j * half:(j + 1) * half] for j in range(fold)],
                         axis=1)
    lanes = fold * w
    tiles = half // 16

    i32 = jax.lax.bitcast_convert_type(xp, jnp.int32)
    s = jnp.where(i32 < 0, i32 ^ jnp.int32(0x7FFFFFFF), i32)

    s_hi = (s >> 16).astype(jnp.int16).reshape(tiles, 16, lanes)
    h = _search16(s_hi, jnp.int32(-32768), k)

    hv = _bcast16(h, lanes)
    m_hi = jnp.where(s_hi > hv, jnp.int16(1), jnp.int16(0))
    cnt_gt = _fold_count(m_hi)
    kp = k - cnt_gt

    z_all = ((s & jnp.int32(0xFFFF)) - jnp.int32(32768)).astype(jnp.int16)
    z = jnp.where(s_hi == hv, z_all.reshape(tiles, 16, lanes),
                  jnp.int16(-32768))
    zstar = _search16(z, jnp.int32(-32768), kp)

    lo = zstar + jnp.int32(32768)
    vstar = (h << 16) | lo
    thr_i = jnp.where(vstar < 0, vstar ^ jnp.int32(0x7FFFFFFF), vstar)
    thr = jax.lax.bitcast_convert_type(thr_i, jnp.float32)
    o_ref[0] = jnp.where(xb >= thr, xb, jnp.float32(0.0)).reshape(xb4.shape)


def kernel(x):
    b = x.shape[0]
    w = x.shape[-1]
    n = x.size // b
    rows = n // w
    fold = max(1, 128 // w)
    while fold > 1 and ((rows % fold) or ((rows // fold) % 16)):
        fold -= 1
    assert rows % 8 == 0 and (rows // fold) % 16 == 0
    block = (1,) + x.shape[1:]
    idx = lambda i: (i,) + (0,) * (x.ndim - 1)
    return pl.pallas_call(
        functools.partial(_topk_mask_kernel, k=_K, fold=fold),
        grid=(b,),
        in_specs=[pl.BlockSpec(block, idx)],
        out_specs=pl.BlockSpec(block, idx),
        out_shape=jax.ShapeDtypeStruct(x.shape, jnp.float32),
        compiler_params=pltpu.CompilerParams(
            dimension_semantics=("parallel",),
        ),
    )(x)


# speedup vs baseline: 2.0802x; 2.0802x over previous
import functools

import jax
import jax.numpy as jnp
from jax.experimental import pallas as pl
from jax.experimental.pallas import tpu as pltpu

_K = 64


def _fold_count(m16):
    a = m16
    n = a.shape[0]
    while n > 1:
        d = next((p for p in (2, 3, 5, 7, 11, 13) if n % p == 0), n)
        step = n // d
        acc = a[0:step]
        for j in range(1, d):
            acc = acc + a[j * step:(j + 1) * step]
        a = acc
        n = step
    return jnp.sum(a[0].astype(jnp.int32))


def _bcast16(t32, lanes):
    return jnp.full((1, 16, lanes), t32, jnp.int32).astype(jnp.int16)


def _count_ge(vlist, t32):
    cnt = jnp.int32(0)
    for v16 in vlist:
        m16 = jnp.where(v16 >= _bcast16(t32, v16.shape[-1]),
                        jnp.int16(1), jnp.int16(0))
        cnt = cnt + _fold_count(m16)
    return cnt


def _search16(vlist, cand0_if_neg, k, nbits=15):
    nonneg = _count_ge(vlist, jnp.int32(0))
    cand = jnp.where(nonneg >= k, jnp.int32(0), cand0_if_neg)
    for b in range(nbits - 1, -1, -1):
        t = cand | jnp.int32(1 << b)
        cnt = _count_ge(vlist, t)
        cand = jnp.where(cnt >= k, t, cand)
    return cand


def _pack_parts(x2):
    rows, ncols = x2.shape
    parts = [x2[:, j * 128:(j + 1) * 128] for j in range(ncols // 128)]
    r = ncols % 128
    if r:
        rem = x2[:, ncols - r:]
        f = 128 // r
        while f > 1 and ((rows % f) or ((rows // f) % 16)):
            f -= 1
        if f > 1:
            h = rows // f
            rem = jnp.concatenate(
                [rem[i * h:(i + 1) * h] for i in range(f)], axis=1)
        parts.append(rem)
    return parts


def _topk_mask_kernel(x_ref, o_ref, *, k):
    xb_nd = x_ref[0]
    ncols = xb_nd.shape[-1]
    rows = xb_nd.size // ncols
    xb = xb_nd.reshape(rows, ncols)

    s_hi_list = []
    s_list = []
    for part in _pack_parts(xb):
        i32 = jax.lax.bitcast_convert_type(part, jnp.int32)
        s = jnp.where(i32 < 0, i32 ^ jnp.int32(0x7FFFFFFF), i32)
        tiles = s.shape[0] // 16
        s_list.append(s)
        s_hi_list.append(
            (s >> 16).astype(jnp.int16).reshape(tiles, 16, s.shape[-1]))

    h = _search16(s_hi_list, jnp.int32(-32768), k)

    cnt_gt = jnp.int32(0)
    z_list = []
    for s, s_hi in zip(s_list, s_hi_list):
        lanes = s.shape[-1]
        hv = _bcast16(h, lanes)
        m_hi = jnp.where(s_hi > hv, jnp.int16(1), jnp.int16(0))
        cnt_gt = cnt_gt + _fold_count(m_hi)
        z_all = ((s & jnp.int32(0xFFFF)) - jnp.int32(32768)).astype(jnp.int16)
        z_list.append(
            jnp.where(s_hi == hv, z_all.reshape(s_hi.shape),
                      jnp.int16(-32768)))
    kp = k - cnt_gt

    zstar = _search16(z_list, jnp.int32(-32768), kp)

    vstar = (h << 16) | (zstar + jnp.int32(32768))
    thr_i = jnp.where(vstar < 0, vstar ^ jnp.int32(0x7FFFFFFF), vstar)
    thr = jax.lax.bitcast_convert_type(thr_i, jnp.float32)
    o_ref[0] = jnp.where(xb >= thr, xb, jnp.float32(0.0)).reshape(xb_nd.shape)


def _run(x):
    b = x.shape[0]
    block = (1,) + x.shape[1:]
    idx = lambda i: (i,) + (0,) * (x.ndim - 1)
    return pl.pallas_call(
        functools.partial(_topk_mask_kernel, k=_K),
        grid=(b,),
        in_specs=[pl.BlockSpec(block, idx)],
        out_specs=pl.BlockSpec(block, idx),
        out_shape=jax.ShapeDtypeStruct(x.shape, jnp.float32),
        compiler_params=pltpu.CompilerParams(
            dimension_semantics=("parallel",),
        ),
    )(x)


def kernel(x):
    if x.ndim == 4 and x.shape[1] > x.shape[-1]:
        out = _run(jnp.transpose(x, (0, 2, 3, 1)))
        return jnp.transpose(out, (0, 3, 1, 2))
    return _run(x)


# speedup vs baseline: 2.1502x; 1.0337x over previous
import functools

import jax
import jax.numpy as jnp
from jax.experimental import pallas as pl
from jax.experimental.pallas import tpu as pltpu

_K = 64


def _fold(a, op):
    n = a.shape[0]
    while n > 1:
        d = next((p for p in (2, 3, 5, 7, 11, 13) if n % p == 0), n)
        step = n // d
        acc = a[0:step]
        for j in range(1, d):
            acc = op(acc, a[j * step:(j + 1) * step])
        a = acc
        n = step
    return a[0]


def _fold_count(m16):
    return jnp.sum(_fold(m16, lambda p, q: p + q).astype(jnp.int32))


def _bcast16(t32, lanes):
    return jnp.full((1, 16, lanes), t32, jnp.int32).astype(jnp.int16)


def _count_ge(vlist, t32):
    cnt = jnp.int32(0)
    for v16 in vlist:
        m16 = jnp.where(v16 >= _bcast16(t32, v16.shape[-1]),
                        jnp.int16(1), jnp.int16(0))
        cnt = cnt + _fold_count(m16)
    return cnt


def _search16(vlist, cand0_if_neg, k, nbits=15):
    nonneg = _count_ge(vlist, jnp.int32(0))
    cand = jnp.where(nonneg >= k, jnp.int32(0), cand0_if_neg)
    for b in range(nbits - 1, -1, -1):
        t = cand | jnp.int32(1 << b)
        cnt = _count_ge(vlist, t)
        cand = jnp.where(cnt >= k, t, cand)
    return cand


def _pack_parts(x2):
    rows, ncols = x2.shape
    parts = [x2[:, j * 128:(j + 1) * 128] for j in range(ncols // 128)]
    r = ncols % 128
    if r:
        rem = x2[:, ncols - r:]
        f = 128 // r
        while f > 1 and ((rows % f) or ((rows // f) % 16)):
            f -= 1
        if f > 1:
            h = rows // f
            rem = jnp.concatenate(
                [rem[i * h:(i + 1) * h] for i in range(f)], axis=1)
        parts.append(rem)
    return parts


def _topk_mask_kernel(x_ref, o_ref, *, k):
    xb_nd = x_ref[0]
    ncols = xb_nd.shape[-1]
    rows = xb_nd.size // ncols
    xb = xb_nd.reshape(rows, ncols)

    s_hi_list = []
    s_list = []
    for part in _pack_parts(xb):
        i32 = jax.lax.bitcast_convert_type(part, jnp.int32)
        s = jnp.where(i32 < 0, i32 ^ jnp.int32(0x7FFFFFFF), i32)
        tiles = s.shape[0] // 16
        s_list.append(s)
        s_hi_list.append(
            (s >> 16).astype(jnp.int16).reshape(tiles, 16, s.shape[-1]))

    h = _search16(s_hi_list, jnp.int32(-32768), k)

    cnt_gt = jnp.int32(0)
    c_eq = jnp.int32(0)
    z_list = []
    zmin_list = []
    for s, s_hi in zip(s_list, s_hi_list):
        lanes = s.shape[-1]
        hv = _bcast16(h, lanes)
        m_hi = jnp.where(s_hi > hv, jnp.int16(1), jnp.int16(0))
        cnt_gt = cnt_gt + _fold_count(m_hi)
        eqm = s_hi == hv
        c_eq = c_eq + _fold_count(
            jnp.where(eqm, jnp.int16(1), jnp.int16(0)))
        z_all = ((s & jnp.int32(0xFFFF)) -
                 jnp.int32(32768)).astype(jnp.int16).reshape(s_hi.shape)
        z_list.append(jnp.where(eqm, z_all, jnp.int16(-32768)))
        zmin_list.append(jnp.where(eqm, z_all, jnp.int16(32767)))
    kp = k - cnt_gt

    def _z_fast():
        m = None
        for zm in zmin_list:
            part_min = jnp.min(_fold(zm.astype(jnp.int32), jnp.minimum))
            m = part_min if m is None else jnp.minimum(m, part_min)
        return m

    zstar = jax.lax.cond(
        cnt_gt + c_eq == k,
        _z_fast,
        lambda: _search16(z_list, jnp.int32(-32768), kp),
    )

    vstar = (h << 16) | (zstar + jnp.int32(32768))
    thr_i = jnp.where(vstar < 0, vstar ^ jnp.int32(0x7FFFFFFF), vstar)
    thr = jax.lax.bitcast_convert_type(thr_i, jnp.float32)
    o_ref[0] = jnp.where(xb >= thr, xb, jnp.float32(0.0)).reshape(xb_nd.shape)


def _run(x):
    b = x.shape[0]
    block = (1,) + x.shape[1:]
    idx = lambda i: (i,) + (0,) * (x.ndim - 1)
    return pl.pallas_call(
        functools.partial(_topk_mask_kernel, k=_K),
        grid=(b,),
        in_specs=[pl.BlockSpec(block, idx)],
        out_specs=pl.BlockSpec(block, idx),
        out_shape=jax.ShapeDtypeStruct(x.shape, jnp.float32),
        compiler_params=pltpu.CompilerParams(
            dimension_semantics=("parallel",),
        ),
    )(x)


def kernel(x):
    if x.ndim == 4 and x.shape[1] > x.shape[-1]:
        out = _run(jnp.transpose(x, (0, 2, 3, 1)))
        return jnp.transpose(out, (0, 3, 1, 2))
    return _run(x)
